# baseline (speedup 1.0000x reference)
import functools

import jax
import jax.numpy as jnp
from jax import lax
from jax.experimental import pallas as pl
from jax.experimental.pallas import tpu as pltpu
from jax.experimental.pallas import tpu_sc as plsc

_EMBED_DIM = 32

_info = plsc.get_sparse_core_info()
_NC, _NS = _info.num_cores, _info.num_subcores
_NW = _NC * _NS

_CHUNK = 1600
_NBUF = 2


def _gather_kernel(n_flat, n_chunks):
    mesh = plsc.VectorSubcoreMesh(core_axis_name="c", subcore_axis_name="s")
    b_per_w = n_flat // _NW

    @functools.partial(
        pl.kernel,
        out_type=jax.ShapeDtypeStruct((n_flat, _EMBED_DIM), jnp.float32),
        mesh=mesh,
        scratch_types=[
            pltpu.VMEM((b_per_w,), jnp.int32),
            pltpu.VMEM((_NBUF, _CHUNK, _EMBED_DIM), jnp.float32),
            pltpu.SemaphoreType.DMA,
            [pltpu.SemaphoreType.DMA] * _NBUF,
            [pltpu.SemaphoreType.DMA] * _NBUF,
        ],
        compiler_params=pltpu.CompilerParams(use_tc_tiling_on_sc=False),
    )
    def k(idx_hbm, table_hbm, out_hbm, idx_v, rows_v, idx_sem, g_sems,
          w_sems):
        wid = lax.axis_index("s") * _NC + lax.axis_index("c")
        base = wid * b_per_w

        def off(i):
            return pl.multiple_of(base + i * _CHUNK, _CHUNK)

        def fire_gather(i):
            b = i % _NBUF
            pltpu.async_copy(
                table_hbm.at[idx_v.at[pl.ds(i * _CHUNK, _CHUNK)]],
                rows_v.at[b], g_sems[b])

        def wait_gather(i):
            b = i % _NBUF
            pltpu.make_async_copy(
                table_hbm.at[idx_v.at[pl.ds(i * _CHUNK, _CHUNK)]],
                rows_v.at[b], g_sems[b]).wait()

        def fire_wb(i):
            b = i % _NBUF
            pltpu.async_copy(rows_v.at[b], out_hbm.at[pl.ds(off(i), _CHUNK)],
                             w_sems[b])

        def wait_wb(i):
            b = i % _NBUF
            pltpu.make_async_copy(rows_v.at[b],
                                  out_hbm.at[pl.ds(off(i), _CHUNK)],
                                  w_sems[b]).wait()

        pltpu.sync_copy(idx_hbm.at[pl.ds(pl.multiple_of(base, _CHUNK),
                                         b_per_w)], idx_v)

        for i in range(n_chunks):
            fire_gather(i)
            if i > 0:
                wait_gather(i - 1)
                fire_wb(i - 1)
                if i + 1 < n_chunks:
                    wait_wb(i - 1)
        wait_gather(n_chunks - 1)
        fire_wb(n_chunks - 1)
        if n_chunks > 1:
            wait_wb(n_chunks - 2)
        wait_wb(n_chunks - 1)

    return k


def kernel(card_indices, table):
    batch, hist = card_indices.shape
    n_flat = batch * hist
    idx_flat = card_indices.reshape(n_flat).astype(jnp.int32)
    n_chunks = n_flat // (_NW * _CHUNK)
    out = _gather_kernel(n_flat, n_chunks)(idx_flat, table)
    return out.reshape(batch, hist, _EMBED_DIM)
